# Initial kernel scaffold; baseline (speedup 1.0000x reference)
import jax
import jax.numpy as jnp
from jax.experimental import pallas as pl


def kernel(x, edge_index, W1, b1, W2, b2, Wf1, bf1, Wf2, bf2, Wf3, bf3):
    raise NotImplementedError("write your pallas kernel here")


# speedup vs baseline: 19.6710x; 19.6710x over previous
import functools

import jax
import jax.numpy as jnp
from jax import lax
from jax.experimental import pallas as pl
from jax.experimental.pallas import tpu as pltpu
from jax.experimental.pallas import tpu_sc as plsc

N_NODES = 10000
NPAD = 10240
IN_DIM = 128
HID = 128
MLP_HID = 256
OUT_DIM = 64
N_EDGES = 320000
CHUNK = 128
NC = 2
NS = 16
NW = NC * NS
CPW = 80
NCHUNKS = NW * CPW
EPAD = NCHUNKS * CHUNK
RPT = NPAD // NS
DEGW = 16


def _sc_mesh():
    return plsc.VectorSubcoreMesh(
        core_axis_name="c", subcore_axis_name="s",
        num_cores=NC, num_subcores=NS)


@functools.cache
def _build_sc_degree():
  @functools.partial(
      pl.kernel,
      out_type=jax.ShapeDtypeStruct((NC * NPAD, DEGW), jnp.float32),
      mesh=_sc_mesh(),
      scratch_types=[
          pltpu.VMEM((CPW, CHUNK), jnp.int32),
          pltpu.VMEM((CHUNK, DEGW), jnp.float32),
          pltpu.VMEM_SHARED((NPAD, DEGW), jnp.float32),
      ],
  )
  def _sc_degree(dst_hbm, zdeg_hbm, out_hbm, idx_v, ones_v, acc_sh):
      c = lax.axis_index("c")
      s = lax.axis_index("s")
      wid = s * NC + c
      pltpu.sync_copy(zdeg_hbm.at[pl.ds(s * RPT, RPT)],
                      acc_sh.at[pl.ds(s * RPT, RPT)])
      def _fill(i, carry):
          ones_v[i] = jnp.ones((DEGW,), jnp.float32)
          return carry
      lax.fori_loop(0, CHUNK, _fill, 0)
      pltpu.sync_copy(dst_hbm.at[pl.ds(wid * CPW, CPW)], idx_v)
      plsc.subcore_barrier()

      def _body(j, carry):
          pltpu.sync_copy(ones_v, acc_sh.at[idx_v.at[j]], add=True)
          return carry
      lax.fori_loop(0, CPW, _body, 0)

      plsc.subcore_barrier()
      pltpu.sync_copy(acc_sh.at[pl.ds(s * RPT, RPT)],
                      out_hbm.at[pl.ds(c * NPAD + s * RPT, RPT)])
  return _sc_degree


@functools.cache
def _build_sc_aggregate():
  @functools.partial(
      pl.kernel,
      out_type=jax.ShapeDtypeStruct((NC * NPAD, HID), jnp.float32),
      mesh=_sc_mesh(),
      scratch_types=[
          pltpu.VMEM((CPW, CHUNK), jnp.int32),
          pltpu.VMEM((CPW, CHUNK), jnp.int32),
          pltpu.VMEM((CHUNK, HID), jnp.float32),
          pltpu.VMEM_SHARED((NPAD, HID), jnp.float32),
          pltpu.SemaphoreType.DMA,
      ],
  )
  def _sc_aggregate(g_hbm, src_hbm, dst_hbm, zero_hbm, out_hbm,
                    src_v, dst_v, rows_v, acc_sh, sem):
      c = lax.axis_index("c")
      s = lax.axis_index("s")
      wid = s * NC + c
      pltpu.sync_copy(zero_hbm.at[pl.ds(s * RPT, RPT)],
                      acc_sh.at[pl.ds(s * RPT, RPT)])
      pltpu.sync_copy(src_hbm.at[pl.ds(wid * CPW, CPW)], src_v)
      pltpu.sync_copy(dst_hbm.at[pl.ds(wid * CPW, CPW)], dst_v)
      plsc.subcore_barrier()

      def _body(j, carry):
          pltpu.async_copy(g_hbm.at[src_v.at[j]], rows_v, sem).wait()
          pltpu.sync_copy(rows_v, acc_sh.at[dst_v.at[j]], add=True)
          return carry
      lax.fori_loop(0, CPW, _body, 0)

      plsc.subcore_barrier()
      pltpu.sync_copy(acc_sh.at[pl.ds(s * RPT, RPT)],
                      out_hbm.at[pl.ds(c * NPAD + s * RPT, RPT)])
  return _sc_aggregate


BLK = 512
GRID = NPAD // BLK


def _dinv_blk(dg0, dg1):
    deg = dg0[:, 0:1] + dg1[:, 0:1] + 1.0
    return lax.rsqrt(deg)


def _tc_g1_body(x_ref, w1_ref, dg0_ref, dg1_ref, g1_ref):
    dinv = _dinv_blk(dg0_ref, dg1_ref)
    g1_ref[...] = jnp.dot(x_ref[...], w1_ref[...],
                          preferred_element_type=jnp.float32) * dinv


def _tc_mid_body(p0_ref, p1_ref, g1_ref, dg0_ref, dg1_ref, b1_ref, w2_ref,
                 g2_ref):
    dinv = _dinv_blk(dg0_ref, dg1_ref)
    h1 = jnp.maximum(
        dinv * (p0_ref[...] + p1_ref[...] + g1_ref[...]) + b1_ref[...], 0.0)
    g2_ref[...] = jnp.dot(h1, w2_ref[...],
                          preferred_element_type=jnp.float32) * dinv


def _tc_final_body(p0_ref, p1_ref, g2_ref, dg0_ref, dg1_ref, b2_ref,
                   wf1_ref, bf1_ref, wf2_ref, bf2_ref, wf3_ref, bf3_ref,
                   out_ref):
    dinv = _dinv_blk(dg0_ref, dg1_ref)
    h2 = jnp.maximum(
        dinv * (p0_ref[...] + p1_ref[...] + g2_ref[...]) + b2_ref[...], 0.0)
    o = jnp.maximum(jnp.dot(h2, wf1_ref[...],
                            preferred_element_type=jnp.float32)
                    + bf1_ref[...], 0.0)
    o = jnp.maximum(jnp.dot(o, wf2_ref[...],
                            preferred_element_type=jnp.float32)
                    + bf2_ref[...], 0.0)
    out_ref[...] = jnp.maximum(jnp.dot(o, wf3_ref[...],
                                       preferred_element_type=jnp.float32)
                               + bf3_ref[...], 0.0)


def _rows(bd):
    return pl.BlockSpec((BLK, bd), lambda i: (i, 0))


def _full(shape):
    return pl.BlockSpec(shape, lambda i: (0,) * len(shape))


def _tc_g1(x, w1, dg0, dg1):
    return pl.pallas_call(
        _tc_g1_body,
        grid=(GRID,),
        in_specs=[_rows(IN_DIM), _full((IN_DIM, HID)), _rows(DEGW),
                  _rows(DEGW)],
        out_specs=_rows(HID),
        out_shape=jax.ShapeDtypeStruct((NPAD, HID), jnp.float32),
    )(x, w1, dg0, dg1)


def _tc_mid(p0, p1, g1, dg0, dg1, b1, w2):
    return pl.pallas_call(
        _tc_mid_body,
        grid=(GRID,),
        in_specs=[_rows(HID), _rows(HID), _rows(HID), _rows(DEGW),
                  _rows(DEGW), _full((1, HID)), _full((HID, HID))],
        out_specs=_rows(HID),
        out_shape=jax.ShapeDtypeStruct((NPAD, HID), jnp.float32),
    )(p0, p1, g1, dg0, dg1, b1, w2)


def _tc_final(p0, p1, g2, dg0, dg1, b2, wf1, bf1, wf2, bf2, wf3, bf3):
    return pl.pallas_call(
        _tc_final_body,
        grid=(GRID,),
        in_specs=[_rows(HID), _rows(HID), _rows(HID), _rows(DEGW),
                  _rows(DEGW), _full((1, HID)),
                  _full((HID, MLP_HID)), _full((1, MLP_HID)),
                  _full((MLP_HID, MLP_HID)), _full((1, MLP_HID)),
                  _full((MLP_HID, OUT_DIM)), _full((1, OUT_DIM))],
        out_specs=_rows(OUT_DIM),
        out_shape=jax.ShapeDtypeStruct((NPAD, OUT_DIM), jnp.float32),
    )(p0, p1, g2, dg0, dg1, b2, wf1, bf1, wf2, bf2, wf3, bf3)


def kernel(x, edge_index, W1, b1, W2, b2, Wf1, bf1, Wf2, bf2, Wf3, bf3):
    f32 = jnp.float32
    ei = edge_index.astype(jnp.int32)
    pad = N_NODES + (jnp.arange(EPAD - N_EDGES, dtype=jnp.int32) % 16)
    src = jnp.concatenate([ei[0], pad]).reshape(NCHUNKS, CHUNK)
    dst = jnp.concatenate([ei[1], pad]).reshape(NCHUNKS, CHUNK)

    xp = jnp.pad(x, ((0, NPAD - N_NODES), (0, 0)))
    zeros_big = jnp.zeros((NPAD, HID), f32)
    zeros_deg = jnp.zeros((NPAD, DEGW), f32)

    sc_degree = _build_sc_degree()
    sc_aggregate = _build_sc_aggregate()

    degp = sc_degree(dst, zeros_deg)
    dg0, dg1 = degp[:NPAD], degp[NPAD:]

    g1 = _tc_g1(xp, W1, dg0, dg1)
    aggp1 = sc_aggregate(g1, src, dst, zeros_big)
    g2 = _tc_mid(aggp1[:NPAD], aggp1[NPAD:], g1, dg0, dg1,
                 b1.reshape(1, HID), W2)
    aggp2 = sc_aggregate(g2, src, dst, zeros_big)
    out = _tc_final(aggp2[:NPAD], aggp2[NPAD:], g2, dg0, dg1,
                    b2.reshape(1, HID), Wf1, bf1.reshape(1, MLP_HID),
                    Wf2, bf2.reshape(1, MLP_HID), Wf3,
                    bf3.reshape(1, OUT_DIM))
    return out[:N_NODES]


# speedup vs baseline: 25.3701x; 1.2897x over previous
import functools

import jax
import jax.numpy as jnp
from jax import lax
from jax.experimental import pallas as pl
from jax.experimental.pallas import tpu as pltpu
from jax.experimental.pallas import tpu_sc as plsc

N_NODES = 10000
NPAD = 10240
IN_DIM = 128
HID = 128
MLP_HID = 256
OUT_DIM = 64
N_EDGES = 320000
CHUNK = 128
NC = 2
NS = 16
NW = NC * NS
CPW = 80
IBLK = 16
NCHUNKS = NW * CPW
EPAD = NCHUNKS * CHUNK
RPT = NPAD // NS
DEGW = 16


def _sc_mesh():
    return plsc.VectorSubcoreMesh(
        core_axis_name="c", subcore_axis_name="s",
        num_cores=NC, num_subcores=NS)


@functools.cache
def _build_sc_degree():
  @functools.partial(
      pl.kernel,
      out_type=jax.ShapeDtypeStruct((NC * NPAD, DEGW), jnp.float32),
      mesh=_sc_mesh(),
      scratch_types=[
          pltpu.VMEM((CPW, CHUNK), jnp.int32),
          pltpu.VMEM((CHUNK, DEGW), jnp.float32),
          pltpu.VMEM_SHARED((NPAD, DEGW), jnp.float32),
      ],
  )
  def _sc_degree(dst_hbm, zdeg_hbm, out_hbm, idx_v, ones_v, acc_sh):
      c = lax.axis_index("c")
      s = lax.axis_index("s")
      wid = s * NC + c
      pltpu.sync_copy(zdeg_hbm.at[pl.ds(s * RPT, RPT)],
                      acc_sh.at[pl.ds(s * RPT, RPT)])
      def _fill(i, carry):
          ones_v[i] = jnp.ones((DEGW,), jnp.float32)
          return carry
      lax.fori_loop(0, CHUNK, _fill, 0)
      pltpu.sync_copy(dst_hbm.at[pl.ds(wid * CPW, CPW)], idx_v)
      plsc.subcore_barrier()

      def _body(j, carry):
          pltpu.sync_copy(ones_v, acc_sh.at[idx_v.at[j]], add=True)
          return carry
      lax.fori_loop(0, CPW, _body, 0)

      plsc.subcore_barrier()
      pltpu.sync_copy(acc_sh.at[pl.ds(s * RPT, RPT)],
                      out_hbm.at[pl.ds(c * NPAD + s * RPT, RPT)])
  return _sc_degree


@functools.cache
def _build_sc_aggregate():
  @functools.partial(
      pl.kernel,
      out_type=jax.ShapeDtypeStruct((NC * NPAD, HID), jnp.float32),
      mesh=_sc_mesh(),
      scratch_types=[
          pltpu.VMEM((IBLK, CHUNK), jnp.int32),
          pltpu.VMEM((IBLK, CHUNK), jnp.int32),
          pltpu.VMEM((CHUNK, HID), jnp.float32),
          pltpu.VMEM((CHUNK, HID), jnp.float32),
          pltpu.VMEM_SHARED((NPAD, HID), jnp.float32),
          pltpu.SemaphoreType.DMA,
          pltpu.SemaphoreType.DMA,
      ],
  )
  def _sc_aggregate(g_hbm, src_hbm, dst_hbm, zero_hbm, out_hbm,
                    src_v, dst_v, rows_a, rows_b, acc_sh, sem_a, sem_b):
      c = lax.axis_index("c")
      s = lax.axis_index("s")
      wid = s * NC + c
      pltpu.sync_copy(zero_hbm.at[pl.ds(s * RPT, RPT)],
                      acc_sh.at[pl.ds(s * RPT, RPT)])
      plsc.subcore_barrier()

      for ib in range(CPW // IBLK):
          base = wid * CPW + ib * IBLK
          pltpu.sync_copy(src_hbm.at[pl.ds(base, IBLK)], src_v)
          pltpu.sync_copy(dst_hbm.at[pl.ds(base, IBLK)], dst_v)
          pltpu.async_copy(g_hbm.at[src_v.at[0]], rows_a, sem_a)

          def _body(j2, carry):
              j = 2 * j2
              pltpu.async_copy(g_hbm.at[src_v.at[j + 1]], rows_b, sem_b)
              pltpu.make_async_copy(g_hbm.at[src_v.at[j]], rows_a,
                                    sem_a).wait()
              pltpu.sync_copy(rows_a, acc_sh.at[dst_v.at[j]], add=True)

              @pl.when(j2 + 1 < IBLK // 2)
              def _():
                  pltpu.async_copy(g_hbm.at[src_v.at[j + 2]], rows_a, sem_a)
              pltpu.make_async_copy(g_hbm.at[src_v.at[j + 1]], rows_b,
                                    sem_b).wait()
              pltpu.sync_copy(rows_b, acc_sh.at[dst_v.at[j + 1]], add=True)
              return carry
          lax.fori_loop(0, IBLK // 2, _body, 0)

      plsc.subcore_barrier()
      pltpu.sync_copy(acc_sh.at[pl.ds(s * RPT, RPT)],
                      out_hbm.at[pl.ds(c * NPAD + s * RPT, RPT)])
  return _sc_aggregate


BLK = 512
GRID = NPAD // BLK


def _dinv_blk(dg0, dg1):
    deg = dg0[:, 0:1] + dg1[:, 0:1] + 1.0
    return lax.rsqrt(deg)


def _tc_g1_body(x_ref, w1_ref, dg0_ref, dg1_ref, g1_ref):
    dinv = _dinv_blk(dg0_ref, dg1_ref)
    g1_ref[...] = jnp.dot(x_ref[...], w1_ref[...],
                          preferred_element_type=jnp.float32) * dinv


def _tc_mid_body(p0_ref, p1_ref, g1_ref, dg0_ref, dg1_ref, b1_ref, w2_ref,
                 g2_ref):
    dinv = _dinv_blk(dg0_ref, dg1_ref)
    h1 = jnp.maximum(
        dinv * (p0_ref[...] + p1_ref[...] + g1_ref[...]) + b1_ref[...], 0.0)
    g2_ref[...] = jnp.dot(h1, w2_ref[...],
                          preferred_element_type=jnp.float32) * dinv


def _tc_final_body(p0_ref, p1_ref, g2_ref, dg0_ref, dg1_ref, b2_ref,
                   wf1_ref, bf1_ref, wf2_ref, bf2_ref, wf3_ref, bf3_ref,
                   out_ref):
    dinv = _dinv_blk(dg0_ref, dg1_ref)
    h2 = jnp.maximum(
        dinv * (p0_ref[...] + p1_ref[...] + g2_ref[...]) + b2_ref[...], 0.0)
    o = jnp.maximum(jnp.dot(h2, wf1_ref[...],
                            preferred_element_type=jnp.float32)
                    + bf1_ref[...], 0.0)
    o = jnp.maximum(jnp.dot(o, wf2_ref[...],
                            preferred_element_type=jnp.float32)
                    + bf2_ref[...], 0.0)
    out_ref[...] = jnp.maximum(jnp.dot(o, wf3_ref[...],
                                       preferred_element_type=jnp.float32)
                               + bf3_ref[...], 0.0)


def _rows(bd):
    return pl.BlockSpec((BLK, bd), lambda i: (i, 0))


def _full(shape):
    return pl.BlockSpec(shape, lambda i: (0,) * len(shape))


def _tc_g1(x, w1, dg0, dg1):
    return pl.pallas_call(
        _tc_g1_body,
        grid=(GRID,),
        in_specs=[_rows(IN_DIM), _full((IN_DIM, HID)), _rows(DEGW),
                  _rows(DEGW)],
        out_specs=_rows(HID),
        out_shape=jax.ShapeDtypeStruct((NPAD, HID), jnp.float32),
    )(x, w1, dg0, dg1)


def _tc_mid(p0, p1, g1, dg0, dg1, b1, w2):
    return pl.pallas_call(
        _tc_mid_body,
        grid=(GRID,),
        in_specs=[_rows(HID), _rows(HID), _rows(HID), _rows(DEGW),
                  _rows(DEGW), _full((1, HID)), _full((HID, HID))],
        out_specs=_rows(HID),
        out_shape=jax.ShapeDtypeStruct((NPAD, HID), jnp.float32),
    )(p0, p1, g1, dg0, dg1, b1, w2)


def _tc_final(p0, p1, g2, dg0, dg1, b2, wf1, bf1, wf2, bf2, wf3, bf3):
    return pl.pallas_call(
        _tc_final_body,
        grid=(GRID,),
        in_specs=[_rows(HID), _rows(HID), _rows(HID), _rows(DEGW),
                  _rows(DEGW), _full((1, HID)),
                  _full((HID, MLP_HID)), _full((1, MLP_HID)),
                  _full((MLP_HID, MLP_HID)), _full((1, MLP_HID)),
                  _full((MLP_HID, OUT_DIM)), _full((1, OUT_DIM))],
        out_specs=_rows(OUT_DIM),
        out_shape=jax.ShapeDtypeStruct((NPAD, OUT_DIM), jnp.float32),
    )(p0, p1, g2, dg0, dg1, b2, wf1, bf1, wf2, bf2, wf3, bf3)


def kernel(x, edge_index, W1, b1, W2, b2, Wf1, bf1, Wf2, bf2, Wf3, bf3):
    f32 = jnp.float32
    ei = edge_index.astype(jnp.int32)
    pad = N_NODES + (jnp.arange(EPAD - N_EDGES, dtype=jnp.int32) % 16)
    src = jnp.concatenate([ei[0], pad]).reshape(NCHUNKS, CHUNK)
    dst = jnp.concatenate([ei[1], pad]).reshape(NCHUNKS, CHUNK)

    xp = jnp.pad(x, ((0, NPAD - N_NODES), (0, 0)))
    zeros_big = jnp.zeros((NPAD, HID), f32)
    zeros_deg = jnp.zeros((NPAD, DEGW), f32)

    sc_degree = _build_sc_degree()
    sc_aggregate = _build_sc_aggregate()

    degp = sc_degree(dst, zeros_deg)
    dg0, dg1 = degp[:NPAD], degp[NPAD:]

    g1 = _tc_g1(xp, W1, dg0, dg1)
    aggp1 = sc_aggregate(g1, src, dst, zeros_big)
    g2 = _tc_mid(aggp1[:NPAD], aggp1[NPAD:], g1, dg0, dg1,
                 b1.reshape(1, HID), W2)
    aggp2 = sc_aggregate(g2, src, dst, zeros_big)
    out = _tc_final(aggp2[:NPAD], aggp2[NPAD:], g2, dg0, dg1,
                    b2.reshape(1, HID), Wf1, bf1.reshape(1, MLP_HID),
                    Wf2, bf2.reshape(1, MLP_HID), Wf3,
                    bf3.reshape(1, OUT_DIM))
    return out[:N_NODES]


# speedup vs baseline: 26.5949x; 1.0483x over previous
import functools

import jax
import jax.numpy as jnp
from jax import lax
from jax.experimental import pallas as pl
from jax.experimental.pallas import tpu as pltpu
from jax.experimental.pallas import tpu_sc as plsc

N_NODES = 10000
NPAD = 10240
IN_DIM = 128
HID = 128
MLP_HID = 256
OUT_DIM = 64
N_EDGES = 320000
CHUNK = 128
NC = 2
NS = 16
NW = NC * NS
CPW = 80
IBLK = 40
NCHUNKS = NW * CPW
EPAD = NCHUNKS * CHUNK
RPT = NPAD // NS
DEGW = 16


def _sc_mesh():
    return plsc.VectorSubcoreMesh(
        core_axis_name="c", subcore_axis_name="s",
        num_cores=NC, num_subcores=NS)


@functools.cache
def _build_sc_degree():
  @functools.partial(
      pl.kernel,
      out_type=jax.ShapeDtypeStruct((NC * NPAD, DEGW), jnp.float32),
      mesh=_sc_mesh(),
      scratch_types=[
          pltpu.VMEM((CPW, CHUNK), jnp.int32),
          pltpu.VMEM((CHUNK, DEGW), jnp.float32),
          pltpu.VMEM_SHARED((NPAD, DEGW), jnp.float32),
      ],
  )
  def _sc_degree(dst_hbm, zdeg_hbm, out_hbm, idx_v, ones_v, acc_sh):
      c = lax.axis_index("c")
      s = lax.axis_index("s")
      wid = s * NC + c
      pltpu.sync_copy(zdeg_hbm.at[pl.ds(s * RPT, RPT)],
                      acc_sh.at[pl.ds(s * RPT, RPT)])
      def _fill(i, carry):
          ones_v[i] = jnp.ones((DEGW,), jnp.float32)
          return carry
      lax.fori_loop(0, CHUNK, _fill, 0)
      pltpu.sync_copy(dst_hbm.at[pl.ds(wid * CPW, CPW)], idx_v)
      plsc.subcore_barrier()

      def _body(j, carry):
          pltpu.sync_copy(ones_v, acc_sh.at[idx_v.at[j]], add=True)
          return carry
      lax.fori_loop(0, CPW, _body, 0)

      plsc.subcore_barrier()
      pltpu.sync_copy(acc_sh.at[pl.ds(s * RPT, RPT)],
                      out_hbm.at[pl.ds(c * NPAD + s * RPT, RPT)])
  return _sc_degree


@functools.cache
def _build_sc_aggregate():
  @functools.partial(
      pl.kernel,
      out_type=jax.ShapeDtypeStruct((NC * NPAD, HID), jnp.float32),
      mesh=_sc_mesh(),
      scratch_types=[
          pltpu.VMEM((IBLK, CHUNK), jnp.int32),
          pltpu.VMEM((IBLK, CHUNK), jnp.int32),
          pltpu.VMEM((CHUNK, HID), jnp.float32),
          pltpu.VMEM((CHUNK, HID), jnp.float32),
          pltpu.VMEM_SHARED((NPAD, HID), jnp.float32),
          pltpu.SemaphoreType.DMA,
          pltpu.SemaphoreType.DMA,
      ],
  )
  def _sc_aggregate(g_hbm, src_hbm, dst_hbm, zero_hbm, out_hbm,
                    src_v, dst_v, rows_a, rows_b, acc_sh, sem_a, sem_b):
      c = lax.axis_index("c")
      s = lax.axis_index("s")
      wid = s * NC + c
      pltpu.sync_copy(zero_hbm.at[pl.ds(s * RPT, RPT)],
                      acc_sh.at[pl.ds(s * RPT, RPT)])
      plsc.subcore_barrier()

      for ib in range(CPW // IBLK):
          base = wid * CPW + ib * IBLK
          pltpu.sync_copy(src_hbm.at[pl.ds(base, IBLK)], src_v)
          pltpu.sync_copy(dst_hbm.at[pl.ds(base, IBLK)], dst_v)
          pltpu.async_copy(g_hbm.at[src_v.at[0]], rows_a, sem_a)

          def _body(j2, carry):
              j = 2 * j2
              pltpu.async_copy(g_hbm.at[src_v.at[j + 1]], rows_b, sem_b)
              pltpu.make_async_copy(g_hbm.at[src_v.at[j]], rows_a,
                                    sem_a).wait()
              pltpu.sync_copy(rows_a, acc_sh.at[dst_v.at[j]], add=True)

              @pl.when(j2 + 1 < IBLK // 2)
              def _():
                  pltpu.async_copy(g_hbm.at[src_v.at[j + 2]], rows_a, sem_a)
              pltpu.make_async_copy(g_hbm.at[src_v.at[j + 1]], rows_b,
                                    sem_b).wait()
              pltpu.sync_copy(rows_b, acc_sh.at[dst_v.at[j + 1]], add=True)
              return carry
          lax.fori_loop(0, IBLK // 2, _body, 0)

      plsc.subcore_barrier()
      pltpu.sync_copy(acc_sh.at[pl.ds(s * RPT, RPT)],
                      out_hbm.at[pl.ds(c * NPAD + s * RPT, RPT)])
  return _sc_aggregate


BLK = 512
GRID = NPAD // BLK


def _dinv_blk(dg0, dg1):
    deg = dg0[:, 0:1] + dg1[:, 0:1] + 1.0
    return lax.rsqrt(deg)


def _tc_g1_body(x_ref, w1_ref, dg0_ref, dg1_ref, g1_ref):
    dinv = _dinv_blk(dg0_ref, dg1_ref)
    g1_ref[...] = jnp.dot(x_ref[...], w1_ref[...],
                          preferred_element_type=jnp.float32) * dinv


def _tc_mid_body(p0_ref, p1_ref, g1_ref, dg0_ref, dg1_ref, b1_ref, w2_ref,
                 g2_ref):
    dinv = _dinv_blk(dg0_ref, dg1_ref)
    h1 = jnp.maximum(
        dinv * (p0_ref[...] + p1_ref[...] + g1_ref[...]) + b1_ref[...], 0.0)
    g2_ref[...] = jnp.dot(h1, w2_ref[...],
                          preferred_element_type=jnp.float32) * dinv


def _tc_final_body(p0_ref, p1_ref, g2_ref, dg0_ref, dg1_ref, b2_ref,
                   wf1_ref, bf1_ref, wf2_ref, bf2_ref, wf3_ref, bf3_ref,
                   out_ref):
    dinv = _dinv_blk(dg0_ref, dg1_ref)
    h2 = jnp.maximum(
        dinv * (p0_ref[...] + p1_ref[...] + g2_ref[...]) + b2_ref[...], 0.0)
    o = jnp.maximum(jnp.dot(h2, wf1_ref[...],
                            preferred_element_type=jnp.float32)
                    + bf1_ref[...], 0.0)
    o = jnp.maximum(jnp.dot(o, wf2_ref[...],
                            preferred_element_type=jnp.float32)
                    + bf2_ref[...], 0.0)
    out_ref[...] = jnp.maximum(jnp.dot(o, wf3_ref[...],
                                       preferred_element_type=jnp.float32)
                               + bf3_ref[...], 0.0)


def _rows(bd):
    return pl.BlockSpec((BLK, bd), lambda i: (i, 0))


def _full(shape):
    return pl.BlockSpec(shape, lambda i: (0,) * len(shape))


def _tc_g1(x, w1, dg0, dg1):
    return pl.pallas_call(
        _tc_g1_body,
        grid=(GRID,),
        in_specs=[_rows(IN_DIM), _full((IN_DIM, HID)), _rows(DEGW),
                  _rows(DEGW)],
        out_specs=_rows(HID),
        out_shape=jax.ShapeDtypeStruct((NPAD, HID), jnp.float32),
    )(x, w1, dg0, dg1)


def _tc_mid(p0, p1, g1, dg0, dg1, b1, w2):
    return pl.pallas_call(
        _tc_mid_body,
        grid=(GRID,),
        in_specs=[_rows(HID), _rows(HID), _rows(HID), _rows(DEGW),
                  _rows(DEGW), _full((1, HID)), _full((HID, HID))],
        out_specs=_rows(HID),
        out_shape=jax.ShapeDtypeStruct((NPAD, HID), jnp.float32),
    )(p0, p1, g1, dg0, dg1, b1, w2)


def _tc_final(p0, p1, g2, dg0, dg1, b2, wf1, bf1, wf2, bf2, wf3, bf3):
    return pl.pallas_call(
        _tc_final_body,
        grid=(GRID,),
        in_specs=[_rows(HID), _rows(HID), _rows(HID), _rows(DEGW),
                  _rows(DEGW), _full((1, HID)),
                  _full((HID, MLP_HID)), _full((1, MLP_HID)),
                  _full((MLP_HID, MLP_HID)), _full((1, MLP_HID)),
                  _full((MLP_HID, OUT_DIM)), _full((1, OUT_DIM))],
        out_specs=_rows(OUT_DIM),
        out_shape=jax.ShapeDtypeStruct((NPAD, OUT_DIM), jnp.float32),
    )(p0, p1, g2, dg0, dg1, b2, wf1, bf1, wf2, bf2, wf3, bf3)


def kernel(x, edge_index, W1, b1, W2, b2, Wf1, bf1, Wf2, bf2, Wf3, bf3):
    f32 = jnp.float32
    ei = edge_index.astype(jnp.int32)
    pad = N_NODES + (jnp.arange(EPAD - N_EDGES, dtype=jnp.int32) % 16)
    src = jnp.concatenate([ei[0], pad]).reshape(NCHUNKS, CHUNK)
    dst = jnp.concatenate([ei[1], pad]).reshape(NCHUNKS, CHUNK)

    xp = jnp.pad(x, ((0, NPAD - N_NODES), (0, 0)))
    zeros_big = jnp.zeros((NPAD, HID), f32)
    zeros_deg = jnp.zeros((NPAD, DEGW), f32)

    sc_degree = _build_sc_degree()
    sc_aggregate = _build_sc_aggregate()

    degp = sc_degree(dst, zeros_deg)
    dg0, dg1 = degp[:NPAD], degp[NPAD:]

    g1 = _tc_g1(xp, W1, dg0, dg1)
    aggp1 = sc_aggregate(g1, src, dst, zeros_big)
    g2 = _tc_mid(aggp1[:NPAD], aggp1[NPAD:], g1, dg0, dg1,
                 b1.reshape(1, HID), W2)
    aggp2 = sc_aggregate(g2, src, dst, zeros_big)
    out = _tc_final(aggp2[:NPAD], aggp2[NPAD:], g2, dg0, dg1,
                    b2.reshape(1, HID), Wf1, bf1.reshape(1, MLP_HID),
                    Wf2, bf2.reshape(1, MLP_HID), Wf3,
                    bf3.reshape(1, OUT_DIM))
    return out[:N_NODES]
